# Initial kernel scaffold; baseline (speedup 1.0000x reference)
import jax
import jax.numpy as jnp
from jax.experimental import pallas as pl


def kernel(node_features, edge_index, Wl, bl, Wr, W1, b1, W2, b2):
    raise NotImplementedError("write your pallas kernel here")


# speedup vs baseline: 2.6731x; 2.6731x over previous
import functools

import jax
import jax.numpy as jnp
from jax import lax
from jax.experimental import pallas as pl
from jax.experimental.pallas import tpu as pltpu
from jax.experimental.pallas import tpu_sc as plsc

N = 10000
E = 320000
D = 128
H = 128

NC = 2
NS = 16
NW = NC * NS
CHUNK = 128
K = (E + NW * CHUNK - 1) // (NW * CHUNK)
E_PAD = NW * CHUNK * K
N_PAD = 10240
ROWS_PER_TILE = N_PAD // NS

_mesh = plsc.VectorSubcoreMesh(core_axis_name="c", subcore_axis_name="s")


@functools.partial(
    pl.kernel,
    out_type=jax.ShapeDtypeStruct((NC, N_PAD, D), jnp.float32),
    mesh=_mesh,
    scratch_types=[
        pltpu.VMEM((K, CHUNK), jnp.int32),
        pltpu.VMEM((K, CHUNK), jnp.int32),
        pltpu.VMEM((CHUNK, D), jnp.float32),
        pltpu.VMEM_SHARED((N_PAD, D), jnp.float32),
        pltpu.SemaphoreType.DMA,
    ],
)
def _sc_segsum(nf_hbm, srcr_hbm, dstr_hbm, zsum_hbm, sums_hbm,
               src_idx, dst_idx, rows, acc_sum, sem):
    c = lax.axis_index("c")
    s = lax.axis_index("s")
    w = c * NS + s

    pltpu.sync_copy(srcr_hbm.at[w], src_idx)
    pltpu.sync_copy(dstr_hbm.at[w], dst_idx)
    r0 = s * ROWS_PER_TILE
    pltpu.sync_copy(zsum_hbm, acc_sum.at[pl.ds(r0, ROWS_PER_TILE)])
    plsc.subcore_barrier()

    def body(j, carry):
        pltpu.async_copy(nf_hbm.at[src_idx.at[j]], rows, sem).wait()
        pltpu.sync_copy(rows, acc_sum.at[dst_idx.at[j]], add=True)
        return carry

    lax.fori_loop(0, K, body, 0)
    plsc.subcore_barrier()

    pltpu.sync_copy(acc_sum.at[pl.ds(r0, ROWS_PER_TILE)],
                    sums_hbm.at[c, pl.ds(r0, ROWS_PER_TILE)])


@functools.partial(
    pl.kernel,
    out_type=jax.ShapeDtypeStruct((NC, N_PAD, D), jnp.float32),
    mesh=_mesh,
    scratch_types=[
        pltpu.VMEM((K, CHUNK), jnp.int32),
        pltpu.VMEM((CHUNK, D), jnp.float32),
        pltpu.VMEM_SHARED((N_PAD, D), jnp.float32),
    ],
)
def _sc_counts(dstr_hbm, zcnt_hbm, ones_hbm, cnts_hbm,
               dst_idx, ones_v, acc_cnt):
    c = lax.axis_index("c")
    s = lax.axis_index("s")
    w = c * NS + s

    pltpu.sync_copy(dstr_hbm.at[w], dst_idx)
    pltpu.sync_copy(ones_hbm, ones_v)
    r0 = s * ROWS_PER_TILE
    pltpu.sync_copy(zcnt_hbm, acc_cnt.at[pl.ds(r0, ROWS_PER_TILE)])
    plsc.subcore_barrier()

    def body(j, carry):
        pltpu.sync_copy(ones_v, acc_cnt.at[dst_idx.at[j]], add=True)
        return carry

    lax.fori_loop(0, K, body, 0)
    plsc.subcore_barrier()

    pltpu.sync_copy(acc_cnt.at[pl.ds(r0, ROWS_PER_TILE)],
                    cnts_hbm.at[c, pl.ds(r0, ROWS_PER_TILE)])


@functools.partial(
    pl.kernel,
    out_type=[
        jax.ShapeDtypeStruct((E_PAD, D), jnp.float32),
        jax.ShapeDtypeStruct((E_PAD, D), jnp.float32),
    ],
    mesh=_mesh,
    scratch_types=[
        pltpu.VMEM((K, CHUNK), jnp.int32),
        pltpu.VMEM((K, CHUNK), jnp.int32),
        pltpu.VMEM((CHUNK, D), jnp.float32),
        pltpu.VMEM((CHUNK, D), jnp.float32),
        pltpu.SemaphoreType.DMA,
        pltpu.SemaphoreType.DMA,
    ],
)
def _sc_gather(table_hbm, srcr_hbm, dstr_hbm, x_hbm, y_hbm,
               src_idx, dst_idx, rx, ry, sem1, sem2):
    c = lax.axis_index("c")
    s = lax.axis_index("s")
    w = c * NS + s

    pltpu.sync_copy(srcr_hbm.at[w], src_idx)
    pltpu.sync_copy(dstr_hbm.at[w], dst_idx)
    base = w * K * CHUNK

    def body(j, carry):
        off = base + j * CHUNK
        cx = pltpu.async_copy(table_hbm.at[src_idx.at[j]], rx, sem1)
        cy = pltpu.async_copy(table_hbm.at[dst_idx.at[j]], ry, sem2)
        cx.wait()
        pltpu.sync_copy(rx, x_hbm.at[pl.ds(off, CHUNK)])
        cy.wait()
        pltpu.sync_copy(ry, y_hbm.at[pl.ds(off, CHUNK)])
        return carry

    lax.fori_loop(0, K, body, 0)


def _tc_node_body(s0, s1, c0, c1, nf, wl, wr, bl, o):
    cnt = jnp.clip(c0[:, :1] + c1[:, :1], 1.0, None)
    mean = (s0[...] + s1[...]) / cnt
    o[...] = jnp.maximum(
        jnp.dot(mean, wl[...], preferred_element_type=jnp.float32)
        + jnp.dot(nf[...], wr[...], preferred_element_type=jnp.float32)
        + bl[...], 0.0)


_BN = 512


def _tc_node(s0, s1, c0, c1, nf_pad, Wl, Wr, bl):
    grid = (N_PAD // _BN,)
    return pl.pallas_call(
        _tc_node_body,
        grid=grid,
        in_specs=[
            pl.BlockSpec((_BN, D), lambda i: (i, 0)),
            pl.BlockSpec((_BN, D), lambda i: (i, 0)),
            pl.BlockSpec((_BN, D), lambda i: (i, 0)),
            pl.BlockSpec((_BN, D), lambda i: (i, 0)),
            pl.BlockSpec((_BN, D), lambda i: (i, 0)),
            pl.BlockSpec((D, H), lambda i: (0, 0)),
            pl.BlockSpec((D, H), lambda i: (0, 0)),
            pl.BlockSpec((1, H), lambda i: (0, 0)),
        ],
        out_specs=pl.BlockSpec((_BN, H), lambda i: (i, 0)),
        out_shape=jax.ShapeDtypeStruct((N_PAD, H), jnp.float32),
    )(s0, s1, c0, c1, nf_pad, Wl, Wr, bl)


def _tc_edge_body(x, y, w1a, w1b, b1, w2, b2, o):
    xv = x[...]
    yv = y[...]
    h = jnp.maximum(
        jnp.dot(xv * yv, w1a[...], preferred_element_type=jnp.float32)
        + jnp.dot(xv - yv, w1b[...], preferred_element_type=jnp.float32)
        + b1[...], 0.0)
    sc = jnp.sum(h * w2[...], axis=1, keepdims=True) + b2[...]
    o[...] = jax.nn.sigmoid(sc)


_BE = 512


def _tc_edge(X, Y, W1a, W1b, b1, w2, b2):
    grid = (E // _BE,)
    return pl.pallas_call(
        _tc_edge_body,
        grid=grid,
        in_specs=[
            pl.BlockSpec((_BE, H), lambda i: (i, 0)),
            pl.BlockSpec((_BE, H), lambda i: (i, 0)),
            pl.BlockSpec((H, H), lambda i: (0, 0)),
            pl.BlockSpec((H, H), lambda i: (0, 0)),
            pl.BlockSpec((1, H), lambda i: (0, 0)),
            pl.BlockSpec((1, H), lambda i: (0, 0)),
            pl.BlockSpec((1, 1), lambda i: (0, 0)),
        ],
        out_specs=pl.BlockSpec((_BE, 1), lambda i: (i, 0)),
        out_shape=jax.ShapeDtypeStruct((E, 1), jnp.float32),
    )(X, Y, W1a, W1b, b1, w2, b2)


def kernel(node_features, edge_index, Wl, bl, Wr, W1, b1, W2, b2):
    src = edge_index[0].astype(jnp.int32)
    dst = edge_index[1].astype(jnp.int32)
    pad = E_PAD - E
    src_p = jnp.concatenate([src, jnp.zeros((pad,), jnp.int32)]).reshape(
        NW, K, CHUNK)
    dst_p = jnp.concatenate([dst, jnp.full((pad,), N, jnp.int32)]).reshape(
        NW, K, CHUNK)

    zsum = jnp.zeros((ROWS_PER_TILE, D), jnp.float32)
    ones = jnp.ones((CHUNK, D), jnp.float32)

    sums = _sc_segsum(node_features, src_p, dst_p, zsum)
    cnts = _sc_counts(dst_p, zsum, ones)

    nf_pad = jnp.pad(node_features, ((0, N_PAD - N), (0, 0)))
    out_table = _tc_node(sums[0], sums[1], cnts[0], cnts[1], nf_pad,
                         Wl, Wr, bl.reshape(1, H))

    X, Y = _sc_gather(out_table, src_p, dst_p)

    prob = _tc_edge(X, Y, W1[:H], W1[H:], b1.reshape(1, H),
                    W2.reshape(1, H), b2.reshape(1, 1))
    return prob
